# Initial kernel scaffold; baseline (speedup 1.0000x reference)
import jax
import jax.numpy as jnp
from jax.experimental import pallas as pl


def kernel(x, edge_index, edge_attr, W_node, b_node, W1_init, W1_root, b1, g1, be1, W2_init, W2_root, b2, g2, be2, W_e1, b_e1, W_e2, b_e2, W_m1, b_m1, W_m2, b_m2):
    raise NotImplementedError("write your pallas kernel here")


# speedup vs baseline: 6.0926x; 6.0926x over previous
import functools

import jax
import jax.numpy as jnp
from jax import lax
from jax.experimental import pallas as pl
from jax.experimental.pallas import tpu as pltpu
from jax.experimental.pallas import tpu_sc as plsc

N = 10000
E = 320000
D_IN = 128
D_EDGE = 16
HID = 50
HP = 128
CP = 64
EPS_BN = 1e-5

NC = 2
NS = 16
NW = NC * NS
EPW = E // NW
CH = 80
NCHUNK = EPW // CH
NP = 10240
RPT = NP // NS

_MESH = plsc.VectorSubcoreMesh(core_axis_name="c", subcore_axis_name="s",
                               num_cores=NC, num_subcores=NS)


def _pad2(w, r, c):
    return jnp.zeros((r, c), jnp.float32).at[:w.shape[0], :w.shape[1]].set(w)


def _pad_row(v, c):
    return jnp.zeros((1, c), jnp.float32).at[0, :v.shape[0]].set(v)


def _hist_body(col_hbm, zero_hbm, out_hbm, col_v, ones_v, deg_sh, sem):
    c = lax.axis_index("c")
    s = lax.axis_index("s")
    wid = s * NC + c
    for i in range(CH // 16):
        ones_v[pl.ds(16 * i, 16)] = jnp.full((16,), 1.0, jnp.float32)
    @pl.when(s == 0)
    def _():
        pltpu.sync_copy(zero_hbm, deg_sh)
    plsc.subcore_barrier()
    base = wid * EPW

    @pl.loop(0, NCHUNK)
    def _(j):
        off = pl.multiple_of(base + j * CH, 8)
        pltpu.sync_copy(col_hbm.at[pl.ds(off, CH)], col_v)
        pltpu.sync_copy(ones_v, deg_sh.at[col_v], add=True)

    plsc.subcore_barrier()
    r0 = s * RPT
    pltpu.sync_copy(deg_sh.at[pl.ds(r0, RPT)], out_hbm.at[c, pl.ds(r0, RPT)])


_hist = pl.kernel(
    _hist_body,
    out_type=jax.ShapeDtypeStruct((NC, NP), jnp.float32),
    mesh=_MESH,
    scratch_types=[
        pltpu.VMEM((CH,), jnp.int32),
        pltpu.VMEM((CH,), jnp.float32),
        pltpu.VMEM_SHARED((NP,), jnp.float32),
        pltpu.SemaphoreType.DMA,
    ],
)


def _segsum_body(u_hbm, row_hbm, col_hbm, zero_hbm, out_hbm,
                 row_v, col_v, gbuf, acc_sh, sem):
    c = lax.axis_index("c")
    s = lax.axis_index("s")
    wid = s * NC + c
    @pl.when(s == 0)
    def _():
        pltpu.sync_copy(zero_hbm, acc_sh)
    plsc.subcore_barrier()
    base = wid * EPW

    @pl.loop(0, NCHUNK)
    def _(j):
        off = pl.multiple_of(base + j * CH, 8)
        pltpu.sync_copy(row_hbm.at[pl.ds(off, CH)], row_v)
        pltpu.sync_copy(col_hbm.at[pl.ds(off, CH)], col_v)
        pltpu.async_copy(u_hbm.at[row_v], gbuf, sem).wait()
        pltpu.sync_copy(gbuf, acc_sh.at[col_v], add=True)

    plsc.subcore_barrier()
    r0 = s * RPT
    pltpu.sync_copy(acc_sh.at[pl.ds(r0, RPT)],
                    out_hbm.at[c, pl.ds(r0, RPT)])


_segsum = pl.kernel(
    _segsum_body,
    out_type=jax.ShapeDtypeStruct((NC, NP, HP), jnp.float32),
    mesh=_MESH,
    scratch_types=[
        pltpu.VMEM((CH,), jnp.int32),
        pltpu.VMEM((CH,), jnp.int32),
        pltpu.VMEM((CH, HP), jnp.float32),
        pltpu.VMEM_SHARED((NP, HP), jnp.float32),
        pltpu.SemaphoreType.DMA,
    ],
)


def _edge_gather_body(a_hbm, b_hbm, row_hbm, col_hbm, ga_hbm, gb_hbm,
                      row_v, col_v, bufa, bufb, sema, semb):
    c = lax.axis_index("c")
    s = lax.axis_index("s")
    wid = s * NC + c
    base = wid * EPW

    @pl.loop(0, NCHUNK)
    def _(j):
        off = pl.multiple_of(base + j * CH, 8)
        pltpu.sync_copy(row_hbm.at[pl.ds(off, CH)], row_v)
        pltpu.sync_copy(col_hbm.at[pl.ds(off, CH)], col_v)
        cpa = pltpu.async_copy(a_hbm.at[row_v], bufa, sema)
        cpb = pltpu.async_copy(b_hbm.at[col_v], bufb, semb)
        cpa.wait()
        cpb.wait()
        pltpu.sync_copy(bufa, ga_hbm.at[pl.ds(off, CH)])
        pltpu.sync_copy(bufb, gb_hbm.at[pl.ds(off, CH)])


_edge_gather = pl.kernel(
    _edge_gather_body,
    out_type=(jax.ShapeDtypeStruct((E, HP), jnp.float32),
              jax.ShapeDtypeStruct((E, HP), jnp.float32)),
    mesh=_MESH,
    scratch_types=[
        pltpu.VMEM((CH,), jnp.int32),
        pltpu.VMEM((CH,), jnp.int32),
        pltpu.VMEM((CH, HP), jnp.float32),
        pltpu.VMEM((CH, HP), jnp.float32),
        pltpu.SemaphoreType.DMA,
        pltpu.SemaphoreType.DMA,
    ],
)


def _tc0_body(x_ref, w_ref, b_ref, o_ref):
    o_ref[...] = jnp.maximum(
        jnp.dot(x_ref[...], w_ref[...], preferred_element_type=jnp.float32)
        + b_ref[...], 0.0)


def _tc1_body(dp_ref, h0_ref, wi_ref, wr_ref, b_ref,
              dis_ref, u_ref, r_ref):
    deg = dp_ref[0, :N, :] + dp_ref[1, :N, :]
    dis = jnp.where(deg > 0, lax.rsqrt(jnp.maximum(deg, 1.0)), 0.0)
    dis_ref[...] = dis
    h0 = h0_ref[...]
    u_ref[...] = dis * jnp.dot(h0, wi_ref[...],
                               preferred_element_type=jnp.float32)
    r_ref[...] = jnp.dot(h0, wr_ref[...],
                         preferred_element_type=jnp.float32) + b_ref[...]


def _bn(t, g, b):
    mu = jnp.mean(t, axis=0, keepdims=True)
    var = jnp.mean((t - mu) ** 2, axis=0, keepdims=True)
    return (t - mu) * lax.rsqrt(var + EPS_BN) * g + b


def _tc2_body(p_ref, r_ref, dis_ref, g_ref, be_ref, wi_ref, wr_ref, b_ref,
              u_ref, r2_ref):
    dis = dis_ref[...]
    agg = dis * (p_ref[0, :N, :] + p_ref[1, :N, :])
    h = _bn(jnp.maximum(agg + r_ref[...], 0.0), g_ref[...], be_ref[...])
    u_ref[...] = dis * jnp.dot(h, wi_ref[...],
                               preferred_element_type=jnp.float32)
    r2_ref[...] = jnp.dot(h, wr_ref[...],
                          preferred_element_type=jnp.float32) + b_ref[...]


def _tc3_body(p_ref, r_ref, dis_ref, g_ref, be_ref, ma_ref, mb_ref,
              a_ref, b_ref):
    agg = dis_ref[...] * (p_ref[0, :N, :] + p_ref[1, :N, :])
    h = _bn(jnp.maximum(agg + r_ref[...], 0.0), g_ref[...], be_ref[...])
    a_ref[...] = jnp.dot(h, ma_ref[...], preferred_element_type=jnp.float32)
    b_ref[...] = jnp.dot(h, mb_ref[...], preferred_element_type=jnp.float32)


def _tc_edgeattr_body(ea_ref, wc_ref, c_ref, o_ref):
    o_ref[...] = jnp.dot(ea_ref[...], wc_ref[...],
                         preferred_element_type=jnp.float32) + c_ref[...]


def _tc_final_body(ga_ref, gb_ref, ce_ref, w2_ref, b2_ref, o_ref):
    g = ga_ref[...] + gb_ref[...]
    t = jnp.tanh(g[:, :CP] + ce_ref[...])
    m = jnp.sum(t * w2_ref[...], axis=1, keepdims=True) + b2_ref[0, 0]
    o_ref[...] = 1.0 / (1.0 + jnp.exp(-m))


def _full(shape, dtype=jnp.float32):
    return pl.BlockSpec(shape, lambda *_: tuple(0 for _ in shape))


BE = 8000


def kernel(x, edge_index, edge_attr, W_node, b_node, W1_init, W1_root, b1,
           g1, be1, W2_init, W2_root, b2, g2, be2, W_e1, b_e1, W_e2, b_e2,
           W_m1, b_m1, W_m2, b_m2):
    f32 = jnp.float32
    row = edge_index[0]
    col = edge_index[1]

    wn = _pad2(W_node, D_IN, HP)
    bn_ = _pad_row(b_node, HP)
    w1i = _pad2(W1_init, HP, HP)
    w1r = _pad2(W1_root, HP, HP)
    b1p = _pad_row(b1, HP)
    g1p = _pad_row(g1, HP)
    be1p = _pad_row(be1, HP)
    w2i = _pad2(W2_init, HP, HP)
    w2r = _pad2(W2_root, HP, HP)
    b2p = _pad_row(b2, HP)
    g2p = _pad_row(g2, HP)
    be2p = _pad_row(be2, HP)
    m1top = W_m1[:HID]
    m1bot = W_m1[HID:]
    ma = _pad2(W_e1[:HID] @ m1top, HP, HP)
    mb = _pad2(W_e1[HID:] @ m1top, HP, HP)
    wc = _pad2(W_e2 @ m1bot, D_EDGE, CP)
    cvec = _pad_row(b_e1 @ m1top + b_e2 @ m1bot + b_m1, CP)
    w2 = _pad_row(W_m2[:, 0], CP)
    b2m = b_m2.reshape(1, 1)

    zero_n = jnp.zeros((NP,), f32)
    zero_nh = jnp.zeros((NP, HP), f32)

    h0 = pl.pallas_call(
        _tc0_body,
        out_shape=jax.ShapeDtypeStruct((N, HP), f32),
    )(x, wn, bn_)

    deg_parts = _hist(col, zero_n)
    deg_parts = deg_parts.reshape(NC, NP, 1)

    dis, u1, r1 = pl.pallas_call(
        _tc1_body,
        out_shape=(jax.ShapeDtypeStruct((N, 1), f32),
                   jax.ShapeDtypeStruct((N, HP), f32),
                   jax.ShapeDtypeStruct((N, HP), f32)),
    )(deg_parts, h0, w1i, w1r, b1p)

    p1 = _segsum(u1, row, col, zero_nh)

    u2, r2 = pl.pallas_call(
        _tc2_body,
        out_shape=(jax.ShapeDtypeStruct((N, HP), f32),
                   jax.ShapeDtypeStruct((N, HP), f32)),
    )(p1, r1, dis, g1p, be1p, w2i, w2r, b2p)

    p2 = _segsum(u2, row, col, zero_nh)

    a2, b2t = pl.pallas_call(
        _tc3_body,
        out_shape=(jax.ShapeDtypeStruct((N, HP), f32),
                   jax.ShapeDtypeStruct((N, HP), f32)),
    )(p2, r2, dis, g2p, be2p, ma, mb)

    ce = pl.pallas_call(
        _tc_edgeattr_body,
        grid=(E // BE,),
        in_specs=[pl.BlockSpec((BE, D_EDGE), lambda i: (i, 0)),
                  _full((D_EDGE, CP)), _full((1, CP))],
        out_specs=pl.BlockSpec((BE, CP), lambda i: (i, 0)),
        out_shape=jax.ShapeDtypeStruct((E, CP), f32),
    )(edge_attr, wc, cvec)

    ga, gb = _edge_gather(a2, b2t, row, col)

    mask = pl.pallas_call(
        _tc_final_body,
        grid=(E // BE,),
        in_specs=[pl.BlockSpec((BE, HP), lambda i: (i, 0)),
                  pl.BlockSpec((BE, HP), lambda i: (i, 0)),
                  pl.BlockSpec((BE, CP), lambda i: (i, 0)),
                  _full((1, CP)), _full((1, 1))],
        out_specs=pl.BlockSpec((BE, 1), lambda i: (i, 0)),
        out_shape=jax.ShapeDtypeStruct((E, 1), f32),
    )(ga, gb, ce, w2, b2m)

    return mask.reshape(-1)


# speedup vs baseline: 6.9503x; 1.1408x over previous
import functools

import jax
import jax.numpy as jnp
from jax import lax
from jax.experimental import pallas as pl
from jax.experimental.pallas import tpu as pltpu
from jax.experimental.pallas import tpu_sc as plsc

N = 10000
E = 320000
D_IN = 128
D_EDGE = 16
HID = 50
HP = 128
CP = 64
EPS_BN = 1e-5

NC = 2
NS = 16
NW = NC * NS
EPW = E // NW
CH = 40
NCHUNK = EPW // CH
CH2 = 80
NCHUNK2 = EPW // CH2
NP = 10240
RPT = NP // NS

_MESH = plsc.VectorSubcoreMesh(core_axis_name="c", subcore_axis_name="s",
                               num_cores=NC, num_subcores=NS)


def _pad2(w, r, c):
    return jnp.zeros((r, c), jnp.float32).at[:w.shape[0], :w.shape[1]].set(w)


def _pad_row(v, c):
    return jnp.zeros((1, c), jnp.float32).at[0, :v.shape[0]].set(v)


def _hist_body(col_hbm, zero_hbm, out_hbm, col_v, ones_v, deg_sh, sem):
    c = lax.axis_index("c")
    s = lax.axis_index("s")
    wid = s * NC + c
    for i in range(CH2 // 16):
        ones_v[pl.ds(16 * i, 16)] = jnp.full((16,), 1.0, jnp.float32)
    @pl.when(s == 0)
    def _():
        pltpu.sync_copy(zero_hbm, deg_sh)
    plsc.subcore_barrier()
    base = wid * EPW

    @pl.loop(0, NCHUNK2)
    def _(j):
        off = pl.multiple_of(base + j * CH2, 8)
        pltpu.sync_copy(col_hbm.at[pl.ds(off, CH2)], col_v)
        pltpu.sync_copy(ones_v, deg_sh.at[col_v], add=True)

    plsc.subcore_barrier()
    r0 = s * RPT
    pltpu.sync_copy(deg_sh.at[pl.ds(r0, RPT)], out_hbm.at[c, pl.ds(r0, RPT)])


_hist = pl.kernel(
    _hist_body,
    out_type=jax.ShapeDtypeStruct((NC, NP), jnp.float32),
    mesh=_MESH,
    scratch_types=[
        pltpu.VMEM((CH2,), jnp.int32),
        pltpu.VMEM((CH2,), jnp.float32),
        pltpu.VMEM_SHARED((NP,), jnp.float32),
        pltpu.SemaphoreType.DMA,
    ],
)


NBUF = 5
NOUT = NCHUNK // NBUF
NOUT2 = NCHUNK2 // NBUF


def _segsum_body(u_hbm, row_hbm, col_hbm, zero_hbm, out_hbm, *sc):
    row_v = sc[0:NBUF]
    col_v = sc[NBUF:2 * NBUF]
    gbuf = sc[2 * NBUF:3 * NBUF]
    acc_sh = sc[3 * NBUF]
    sga = sc[3 * NBUF + 1:4 * NBUF + 1]
    ssc = sc[4 * NBUF + 1:5 * NBUF + 1]
    c = lax.axis_index("c")
    s = lax.axis_index("s")
    wid = s * NC + c
    @pl.when(s == 0)
    def _():
        pltpu.sync_copy(zero_hbm, acc_sh)
    plsc.subcore_barrier()
    base = wid * EPW

    for b in range(NBUF):
        off = pl.multiple_of(base + b * CH, 8)
        pltpu.sync_copy(row_hbm.at[pl.ds(off, CH)], row_v[b])
        pltpu.sync_copy(col_hbm.at[pl.ds(off, CH)], col_v[b])
        pltpu.async_copy(u_hbm.at[row_v[b]], gbuf[b], sga[b])

    @pl.loop(0, NOUT)
    def _(g):
        for b in range(NBUF):
            pltpu.make_async_copy(u_hbm.at[row_v[b]], gbuf[b], sga[b]).wait()
            pltpu.async_copy(gbuf[b], acc_sh.at[col_v[b]], ssc[b], add=True)
        for b in range(NBUF):
            pltpu.make_async_copy(gbuf[b], acc_sh.at[col_v[b]], ssc[b]).wait()
            @pl.when(g < NOUT - 1)
            def _():
                off = pl.multiple_of(base + ((g + 1) * NBUF + b) * CH, 8)
                pltpu.sync_copy(row_hbm.at[pl.ds(off, CH)], row_v[b])
                pltpu.sync_copy(col_hbm.at[pl.ds(off, CH)], col_v[b])
                pltpu.async_copy(u_hbm.at[row_v[b]], gbuf[b], sga[b])

    plsc.subcore_barrier()
    r0 = s * RPT
    pltpu.sync_copy(acc_sh.at[pl.ds(r0, RPT)],
                    out_hbm.at[c, pl.ds(r0, RPT)])


_segsum = pl.kernel(
    _segsum_body,
    out_type=jax.ShapeDtypeStruct((NC, NP, HP), jnp.float32),
    mesh=_MESH,
    scratch_types=(
        [pltpu.VMEM((CH,), jnp.int32) for _ in range(2 * NBUF)]
        + [pltpu.VMEM((CH, HP), jnp.float32) for _ in range(NBUF)]
        + [pltpu.VMEM_SHARED((NP, HP), jnp.float32)]
        + [pltpu.SemaphoreType.DMA for _ in range(2 * NBUF)]
    ),
)


def _edge_gather_body(a_hbm, b_hbm, row_hbm, col_hbm, ga_hbm, gb_hbm, *sc):
    row_v = sc[0:NBUF]
    col_v = sc[NBUF:2 * NBUF]
    bufa = sc[2 * NBUF:3 * NBUF]
    bufb = sc[3 * NBUF:4 * NBUF]
    sga = sc[4 * NBUF:5 * NBUF]
    sgb = sc[5 * NBUF:6 * NBUF]
    swa = sc[6 * NBUF:7 * NBUF]
    swb = sc[7 * NBUF:8 * NBUF]
    c = lax.axis_index("c")
    s = lax.axis_index("s")
    wid = s * NC + c
    base = wid * EPW

    for b in range(NBUF):
        off = pl.multiple_of(base + b * CH2, 8)
        pltpu.sync_copy(row_hbm.at[pl.ds(off, CH2)], row_v[b])
        pltpu.sync_copy(col_hbm.at[pl.ds(off, CH2)], col_v[b])
        pltpu.async_copy(a_hbm.at[row_v[b]], bufa[b], sga[b])
        pltpu.async_copy(b_hbm.at[col_v[b]], bufb[b], sgb[b])

    @pl.loop(0, NOUT2)
    def _(g):
        for b in range(NBUF):
            off = pl.multiple_of(base + (g * NBUF + b) * CH2, 8)
            pltpu.make_async_copy(a_hbm.at[row_v[b]], bufa[b], sga[b]).wait()
            pltpu.async_copy(bufa[b], ga_hbm.at[pl.ds(off, CH2)], swa[b])
            pltpu.make_async_copy(b_hbm.at[col_v[b]], bufb[b], sgb[b]).wait()
            pltpu.async_copy(bufb[b], gb_hbm.at[pl.ds(off, CH2)], swb[b])
        for b in range(NBUF):
            off = pl.multiple_of(base + (g * NBUF + b) * CH2, 8)
            pltpu.make_async_copy(bufa[b], ga_hbm.at[pl.ds(off, CH2)],
                                  swa[b]).wait()
            pltpu.make_async_copy(bufb[b], gb_hbm.at[pl.ds(off, CH2)],
                                  swb[b]).wait()
            @pl.when(g < NOUT2 - 1)
            def _():
                off2 = pl.multiple_of(base + ((g + 1) * NBUF + b) * CH2, 8)
                pltpu.sync_copy(row_hbm.at[pl.ds(off2, CH2)], row_v[b])
                pltpu.sync_copy(col_hbm.at[pl.ds(off2, CH2)], col_v[b])
                pltpu.async_copy(a_hbm.at[row_v[b]], bufa[b], sga[b])
                pltpu.async_copy(b_hbm.at[col_v[b]], bufb[b], sgb[b])


_edge_gather = pl.kernel(
    _edge_gather_body,
    out_type=(jax.ShapeDtypeStruct((E, HP), jnp.float32),
              jax.ShapeDtypeStruct((E, HP), jnp.float32)),
    mesh=_MESH,
    scratch_types=(
        [pltpu.VMEM((CH2,), jnp.int32) for _ in range(2 * NBUF)]
        + [pltpu.VMEM((CH2, HP), jnp.float32) for _ in range(2 * NBUF)]
        + [pltpu.SemaphoreType.DMA for _ in range(4 * NBUF)]
    ),
)


def _tc0_body(x_ref, w_ref, b_ref, o_ref):
    o_ref[...] = jnp.maximum(
        jnp.dot(x_ref[...], w_ref[...], preferred_element_type=jnp.float32)
        + b_ref[...], 0.0)


def _tc1_body(dp_ref, h0_ref, wi_ref, wr_ref, b_ref,
              dis_ref, u_ref, r_ref):
    deg = dp_ref[0, :N, :] + dp_ref[1, :N, :]
    dis = jnp.where(deg > 0, lax.rsqrt(jnp.maximum(deg, 1.0)), 0.0)
    dis_ref[...] = dis
    h0 = h0_ref[...]
    u_ref[...] = dis * jnp.dot(h0, wi_ref[...],
                               preferred_element_type=jnp.float32)
    r_ref[...] = jnp.dot(h0, wr_ref[...],
                         preferred_element_type=jnp.float32) + b_ref[...]


def _bn(t, g, b):
    mu = jnp.mean(t, axis=0, keepdims=True)
    var = jnp.mean((t - mu) ** 2, axis=0, keepdims=True)
    return (t - mu) * lax.rsqrt(var + EPS_BN) * g + b


def _tc2_body(p_ref, r_ref, dis_ref, g_ref, be_ref, wi_ref, wr_ref, b_ref,
              u_ref, r2_ref):
    dis = dis_ref[...]
    agg = dis * (p_ref[0, :N, :] + p_ref[1, :N, :])
    h = _bn(jnp.maximum(agg + r_ref[...], 0.0), g_ref[...], be_ref[...])
    u_ref[...] = dis * jnp.dot(h, wi_ref[...],
                               preferred_element_type=jnp.float32)
    r2_ref[...] = jnp.dot(h, wr_ref[...],
                          preferred_element_type=jnp.float32) + b_ref[...]


def _tc3_body(p_ref, r_ref, dis_ref, g_ref, be_ref, ma_ref, mb_ref,
              a_ref, b_ref):
    agg = dis_ref[...] * (p_ref[0, :N, :] + p_ref[1, :N, :])
    h = _bn(jnp.maximum(agg + r_ref[...], 0.0), g_ref[...], be_ref[...])
    a_ref[...] = jnp.dot(h, ma_ref[...], preferred_element_type=jnp.float32)
    b_ref[...] = jnp.dot(h, mb_ref[...], preferred_element_type=jnp.float32)


def _tc_edgeattr_body(ea_ref, wc_ref, c_ref, o_ref):
    o_ref[...] = jnp.dot(ea_ref[...], wc_ref[...],
                         preferred_element_type=jnp.float32) + c_ref[...]


def _tc_final_body(ga_ref, gb_ref, ce_ref, w2_ref, b2_ref, o_ref):
    g = ga_ref[...] + gb_ref[...]
    t = jnp.tanh(g[:, :CP] + ce_ref[...])
    m = jnp.sum(t * w2_ref[...], axis=1, keepdims=True) + b2_ref[0, 0]
    o_ref[...] = 1.0 / (1.0 + jnp.exp(-m))


def _full(shape, dtype=jnp.float32):
    return pl.BlockSpec(shape, lambda *_: tuple(0 for _ in shape))


BE = 8000


def kernel(x, edge_index, edge_attr, W_node, b_node, W1_init, W1_root, b1,
           g1, be1, W2_init, W2_root, b2, g2, be2, W_e1, b_e1, W_e2, b_e2,
           W_m1, b_m1, W_m2, b_m2):
    f32 = jnp.float32
    row = edge_index[0]
    col = edge_index[1]

    wn = _pad2(W_node, D_IN, HP)
    bn_ = _pad_row(b_node, HP)
    w1i = _pad2(W1_init, HP, HP)
    w1r = _pad2(W1_root, HP, HP)
    b1p = _pad_row(b1, HP)
    g1p = _pad_row(g1, HP)
    be1p = _pad_row(be1, HP)
    w2i = _pad2(W2_init, HP, HP)
    w2r = _pad2(W2_root, HP, HP)
    b2p = _pad_row(b2, HP)
    g2p = _pad_row(g2, HP)
    be2p = _pad_row(be2, HP)
    m1top = W_m1[:HID]
    m1bot = W_m1[HID:]
    ma = _pad2(W_e1[:HID] @ m1top, HP, HP)
    mb = _pad2(W_e1[HID:] @ m1top, HP, HP)
    wc = _pad2(W_e2 @ m1bot, D_EDGE, CP)
    cvec = _pad_row(b_e1 @ m1top + b_e2 @ m1bot + b_m1, CP)
    w2 = _pad_row(W_m2[:, 0], CP)
    b2m = b_m2.reshape(1, 1)

    zero_n = jnp.zeros((NP,), f32)
    zero_nh = jnp.zeros((NP, HP), f32)

    h0 = pl.pallas_call(
        _tc0_body,
        out_shape=jax.ShapeDtypeStruct((N, HP), f32),
    )(x, wn, bn_)

    deg_parts = _hist(col, zero_n)
    deg_parts = deg_parts.reshape(NC, NP, 1)

    dis, u1, r1 = pl.pallas_call(
        _tc1_body,
        out_shape=(jax.ShapeDtypeStruct((N, 1), f32),
                   jax.ShapeDtypeStruct((N, HP), f32),
                   jax.ShapeDtypeStruct((N, HP), f32)),
    )(deg_parts, h0, w1i, w1r, b1p)

    p1 = _segsum(u1, row, col, zero_nh)

    u2, r2 = pl.pallas_call(
        _tc2_body,
        out_shape=(jax.ShapeDtypeStruct((N, HP), f32),
                   jax.ShapeDtypeStruct((N, HP), f32)),
    )(p1, r1, dis, g1p, be1p, w2i, w2r, b2p)

    p2 = _segsum(u2, row, col, zero_nh)

    a2, b2t = pl.pallas_call(
        _tc3_body,
        out_shape=(jax.ShapeDtypeStruct((N, HP), f32),
                   jax.ShapeDtypeStruct((N, HP), f32)),
    )(p2, r2, dis, g2p, be2p, ma, mb)

    ce = pl.pallas_call(
        _tc_edgeattr_body,
        grid=(E // BE,),
        in_specs=[pl.BlockSpec((BE, D_EDGE), lambda i: (i, 0)),
                  _full((D_EDGE, CP)), _full((1, CP))],
        out_specs=pl.BlockSpec((BE, CP), lambda i: (i, 0)),
        out_shape=jax.ShapeDtypeStruct((E, CP), f32),
    )(edge_attr, wc, cvec)

    ga, gb = _edge_gather(a2, b2t, row, col)

    mask = pl.pallas_call(
        _tc_final_body,
        grid=(E // BE,),
        in_specs=[pl.BlockSpec((BE, HP), lambda i: (i, 0)),
                  pl.BlockSpec((BE, HP), lambda i: (i, 0)),
                  pl.BlockSpec((BE, CP), lambda i: (i, 0)),
                  _full((1, CP)), _full((1, 1))],
        out_specs=pl.BlockSpec((BE, 1), lambda i: (i, 0)),
        out_shape=jax.ShapeDtypeStruct((E, 1), f32),
    )(ga, gb, ce, w2, b2m)

    return mask.reshape(-1)


# speedup vs baseline: 7.5458x; 1.0857x over previous
import jax
import jax.numpy as jnp
from jax import lax
from jax.experimental import pallas as pl
from jax.experimental.pallas import tpu as pltpu
from jax.experimental.pallas import tpu_sc as plsc

N = 10000
E = 320000
D_IN = 128
D_EDGE = 16
HID = 50
HP = 128
HQ = 64
EPS_BN = 1e-5

NC = 2
NS = 16
NW = NC * NS
EPW = E // NW
NP = 10240
RPT = NP // NS

CH = 40
NCHUNK = EPW // CH
NBUF = 5
NOUT = NCHUNK // NBUF

CH2 = 40
NCHUNK2 = EPW // CH2
EBUF = 5
NOUT2 = NCHUNK2 // EBUF

CHH = 80
NCHUNKH = EPW // CHH

_MESH = plsc.VectorSubcoreMesh(core_axis_name="c", subcore_axis_name="s",
                               num_cores=NC, num_subcores=NS)


def _pad2(w, r, c):
    return jnp.zeros((r, c), jnp.float32).at[:w.shape[0], :w.shape[1]].set(w)


def _pad_row(v, c):
    return jnp.zeros((1, c), jnp.float32).at[0, :v.shape[0]].set(v)


def _hist_body(col_hbm, zero_hbm, out_hbm, col_v, ones_v, deg_sh, sem):
    c = lax.axis_index("c")
    s = lax.axis_index("s")
    wid = s * NC + c
    for i in range(CHH // 16):
        ones_v[pl.ds(16 * i, 16)] = jnp.full((16,), 1.0, jnp.float32)
    @pl.when(s == 0)
    def _():
        pltpu.sync_copy(zero_hbm, deg_sh)
    plsc.subcore_barrier()
    base = wid * EPW

    @pl.loop(0, NCHUNKH)
    def _(j):
        off = pl.multiple_of(base + j * CHH, 8)
        pltpu.sync_copy(col_hbm.at[pl.ds(off, CHH)], col_v)
        pltpu.sync_copy(ones_v, deg_sh.at[col_v], add=True)

    plsc.subcore_barrier()
    r0 = s * RPT
    pltpu.sync_copy(deg_sh.at[pl.ds(r0, RPT)], out_hbm.at[c, pl.ds(r0, RPT)])


_hist = pl.kernel(
    _hist_body,
    out_type=jax.ShapeDtypeStruct((NC, NP), jnp.float32),
    mesh=_MESH,
    scratch_types=[
        pltpu.VMEM((CHH,), jnp.int32),
        pltpu.VMEM((CHH,), jnp.float32),
        pltpu.VMEM_SHARED((NP,), jnp.float32),
        pltpu.SemaphoreType.DMA,
    ],
)


def _segsum_body(u_hbm, row_hbm, col_hbm, zero_hbm, out_hbm, *sc):
    row_v = sc[0:NBUF]
    col_v = sc[NBUF:2 * NBUF]
    gbuf = sc[2 * NBUF:3 * NBUF]
    acc_sh = sc[3 * NBUF]
    sga = sc[3 * NBUF + 1:4 * NBUF + 1]
    ssc = sc[4 * NBUF + 1:5 * NBUF + 1]
    c = lax.axis_index("c")
    s = lax.axis_index("s")
    wid = s * NC + c
    @pl.when(s == 0)
    def _():
        pltpu.sync_copy(zero_hbm, acc_sh)
    plsc.subcore_barrier()
    base = wid * EPW

    for b in range(NBUF):
        off = pl.multiple_of(base + b * CH, 8)
        pltpu.sync_copy(row_hbm.at[pl.ds(off, CH)], row_v[b])
        pltpu.sync_copy(col_hbm.at[pl.ds(off, CH)], col_v[b])
        pltpu.async_copy(u_hbm.at[row_v[b]], gbuf[b], sga[b])

    @pl.loop(0, NOUT)
    def _(g):
        for b in range(NBUF):
            pltpu.make_async_copy(u_hbm.at[row_v[b]], gbuf[b], sga[b]).wait()
            pltpu.async_copy(gbuf[b], acc_sh.at[col_v[b]], ssc[b], add=True)
        for b in range(NBUF):
            pltpu.make_async_copy(gbuf[b], acc_sh.at[col_v[b]], ssc[b]).wait()
            @pl.when(g < NOUT - 1)
            def _():
                off = pl.multiple_of(base + ((g + 1) * NBUF + b) * CH, 8)
                pltpu.sync_copy(row_hbm.at[pl.ds(off, CH)], row_v[b])
                pltpu.sync_copy(col_hbm.at[pl.ds(off, CH)], col_v[b])
                pltpu.async_copy(u_hbm.at[row_v[b]], gbuf[b], sga[b])

    plsc.subcore_barrier()
    r0 = s * RPT
    pltpu.sync_copy(acc_sh.at[pl.ds(r0, RPT)],
                    out_hbm.at[c, pl.ds(r0, RPT)])


_segsum = pl.kernel(
    _segsum_body,
    out_type=jax.ShapeDtypeStruct((NC, NP, HP), jnp.float32),
    mesh=_MESH,
    scratch_types=(
        [pltpu.VMEM((CH,), jnp.int32) for _ in range(2 * NBUF)]
        + [pltpu.VMEM((CH, HP), jnp.float32) for _ in range(NBUF)]
        + [pltpu.VMEM_SHARED((NP, HP), jnp.float32)]
        + [pltpu.SemaphoreType.DMA for _ in range(2 * NBUF)]
    ),
)


def _edge_body(a_hbm, b_hbm, row_hbm, col_hbm, g_hbm, *sc):
    row_v = sc[0:EBUF]
    col_v = sc[EBUF:2 * EBUF]
    bufa = sc[2 * EBUF:3 * EBUF]
    bufb = sc[3 * EBUF:4 * EBUF]
    bufc = sc[4 * EBUF:5 * EBUF]
    sga = sc[5 * EBUF:6 * EBUF]
    sgb = sc[6 * EBUF:7 * EBUF]
    swc = sc[7 * EBUF:8 * EBUF]
    c = lax.axis_index("c")
    s = lax.axis_index("s")
    wid = s * NC + c
    base = wid * EPW

    for b in range(EBUF):
        off = pl.multiple_of(base + b * CH2, 8)
        pltpu.sync_copy(row_hbm.at[pl.ds(off, CH2)], row_v[b])
        pltpu.sync_copy(col_hbm.at[pl.ds(off, CH2)], col_v[b])
        pltpu.async_copy(a_hbm.at[row_v[b]], bufa[b], sga[b])
        pltpu.async_copy(b_hbm.at[col_v[b]], bufb[b], sgb[b])

    @pl.loop(0, NOUT2)
    def _(g):
        for b in range(EBUF):
            off = pl.multiple_of(base + (g * EBUF + b) * CH2, 8)
            pltpu.make_async_copy(a_hbm.at[row_v[b]], bufa[b], sga[b]).wait()
            pltpu.make_async_copy(b_hbm.at[col_v[b]], bufb[b], sgb[b]).wait()

            @pl.loop(0, CH2)
            def _(r):
                for k in range(HQ // 16):
                    bufc[b][r, pl.ds(16 * k, 16)] = (
                        bufa[b][r, pl.ds(16 * k, 16)]
                        + bufb[b][r, pl.ds(16 * k, 16)])

            pltpu.async_copy(bufc[b], g_hbm.at[pl.ds(off, CH2)], swc[b])
        for b in range(EBUF):
            off = pl.multiple_of(base + (g * EBUF + b) * CH2, 8)
            pltpu.make_async_copy(bufc[b], g_hbm.at[pl.ds(off, CH2)],
                                  swc[b]).wait()
            @pl.when(g < NOUT2 - 1)
            def _():
                off2 = pl.multiple_of(base + ((g + 1) * EBUF + b) * CH2, 8)
                pltpu.sync_copy(row_hbm.at[pl.ds(off2, CH2)], row_v[b])
                pltpu.sync_copy(col_hbm.at[pl.ds(off2, CH2)], col_v[b])
                pltpu.async_copy(a_hbm.at[row_v[b]], bufa[b], sga[b])
                pltpu.async_copy(b_hbm.at[col_v[b]], bufb[b], sgb[b])


_edge = pl.kernel(
    _edge_body,
    out_type=jax.ShapeDtypeStruct((E, HQ), jnp.float32),
    mesh=_MESH,
    scratch_types=(
        [pltpu.VMEM((CH2,), jnp.int32) for _ in range(2 * EBUF)]
        + [pltpu.VMEM((CH2, HP), jnp.float32) for _ in range(2 * EBUF)]
        + [pltpu.VMEM((CH2, HQ), jnp.float32) for _ in range(EBUF)]
        + [pltpu.SemaphoreType.DMA for _ in range(3 * EBUF)]
    ),
)


def _tc0_body(x_ref, w_ref, b_ref, o_ref):
    o_ref[...] = jnp.maximum(
        jnp.dot(x_ref[...], w_ref[...], preferred_element_type=jnp.float32)
        + b_ref[...], 0.0)


def _tc1_body(dp_ref, h0_ref, wi_ref, wr_ref, b_ref,
              dis_ref, u_ref, r_ref):
    deg = dp_ref[0, :N, :] + dp_ref[1, :N, :]
    dis = jnp.where(deg > 0, lax.rsqrt(jnp.maximum(deg, 1.0)), 0.0)
    dis_ref[...] = dis
    h0 = h0_ref[...]
    u_ref[...] = dis * jnp.dot(h0, wi_ref[...],
                               preferred_element_type=jnp.float32)
    r_ref[...] = jnp.dot(h0, wr_ref[...],
                         preferred_element_type=jnp.float32) + b_ref[...]


def _bn(t, g, b):
    mu = jnp.mean(t, axis=0, keepdims=True)
    var = jnp.mean((t - mu) ** 2, axis=0, keepdims=True)
    return (t - mu) * lax.rsqrt(var + EPS_BN) * g + b


def _tc2_body(p_ref, r_ref, dis_ref, g_ref, be_ref, wi_ref, wr_ref, b_ref,
              u_ref, r2_ref):
    dis = dis_ref[...]
    agg = dis * (p_ref[0, :N, :HQ] + p_ref[1, :N, :HQ])
    h = _bn(jnp.maximum(agg + r_ref[...], 0.0), g_ref[...], be_ref[...])
    u_ref[...] = dis * jnp.dot(h, wi_ref[...],
                               preferred_element_type=jnp.float32)
    r2_ref[...] = jnp.dot(h, wr_ref[...],
                          preferred_element_type=jnp.float32) + b_ref[...]


def _tc3_body(p_ref, r_ref, dis_ref, g_ref, be_ref, ma_ref, mb_ref,
              a_ref, b_ref):
    agg = dis_ref[...] * (p_ref[0, :N, :HQ] + p_ref[1, :N, :HQ])
    h = _bn(jnp.maximum(agg + r_ref[...], 0.0), g_ref[...], be_ref[...])
    a_ref[...] = jnp.dot(h, ma_ref[...], preferred_element_type=jnp.float32)
    b_ref[...] = jnp.dot(h, mb_ref[...], preferred_element_type=jnp.float32)


def _tc_final_body(g_ref, ea_ref, wc_ref, c_ref, w2_ref, b2_ref, o_ref):
    ce = jnp.dot(ea_ref[...], wc_ref[...],
                 preferred_element_type=jnp.float32) + c_ref[...]
    t = jnp.tanh(g_ref[...] + ce)
    m = jnp.sum(t * w2_ref[...], axis=1, keepdims=True) + b2_ref[0, 0]
    o_ref[...] = 1.0 / (1.0 + jnp.exp(-m))


def _full(shape, dtype=jnp.float32):
    return pl.BlockSpec(shape, lambda *_: tuple(0 for _ in shape))


BE = 8000


def kernel(x, edge_index, edge_attr, W_node, b_node, W1_init, W1_root, b1,
           g1, be1, W2_init, W2_root, b2, g2, be2, W_e1, b_e1, W_e2, b_e2,
           W_m1, b_m1, W_m2, b_m2):
    f32 = jnp.float32
    row = edge_index[0]
    col = edge_index[1]

    wn = _pad2(W_node, D_IN, HQ)
    bn_ = _pad_row(b_node, HQ)
    w1i = _pad2(W1_init, HQ, HP)
    w1r = _pad2(W1_root, HQ, HQ)
    b1p = _pad_row(b1, HQ)
    g1p = _pad_row(g1, HQ)
    be1p = _pad_row(be1, HQ)
    w2i = _pad2(W2_init, HQ, HP)
    w2r = _pad2(W2_root, HQ, HQ)
    b2p = _pad_row(b2, HQ)
    g2p = _pad_row(g2, HQ)
    be2p = _pad_row(be2, HQ)
    m1top = W_m1[:HID]
    m1bot = W_m1[HID:]
    ma = _pad2(W_e1[:HID] @ m1top, HQ, HP)
    mb = _pad2(W_e1[HID:] @ m1top, HQ, HP)
    wc = _pad2(W_e2 @ m1bot, D_EDGE, HQ)
    cvec = _pad_row(b_e1 @ m1top + b_e2 @ m1bot + b_m1, HQ)
    w2 = _pad_row(W_m2[:, 0], HQ)
    b2m = b_m2.reshape(1, 1)

    zero_n = jnp.zeros((NP,), f32)
    zero_nh = jnp.zeros((NP, HP), f32)

    h0 = pl.pallas_call(
        _tc0_body,
        out_shape=jax.ShapeDtypeStruct((N, HQ), f32),
    )(x, wn, bn_)

    deg_parts = _hist(col, zero_n)
    deg_parts = deg_parts.reshape(NC, NP, 1)

    dis, u1, r1 = pl.pallas_call(
        _tc1_body,
        out_shape=(jax.ShapeDtypeStruct((N, 1), f32),
                   jax.ShapeDtypeStruct((N, HP), f32),
                   jax.ShapeDtypeStruct((N, HQ), f32)),
    )(deg_parts, h0, w1i, w1r, b1p)

    p1 = _segsum(u1, row, col, zero_nh)

    u2, r2 = pl.pallas_call(
        _tc2_body,
        out_shape=(jax.ShapeDtypeStruct((N, HP), f32),
                   jax.ShapeDtypeStruct((N, HQ), f32)),
    )(p1, r1, dis, g1p, be1p, w2i, w2r, b2p)

    p2 = _segsum(u2, row, col, zero_nh)

    a2, b2t = pl.pallas_call(
        _tc3_body,
        out_shape=(jax.ShapeDtypeStruct((N, HP), f32),
                   jax.ShapeDtypeStruct((N, HP), f32)),
    )(p2, r2, dis, g2p, be2p, ma, mb)

    gsum = _edge(a2, b2t, row, col)

    mask = pl.pallas_call(
        _tc_final_body,
        grid=(E // BE,),
        in_specs=[pl.BlockSpec((BE, HQ), lambda i: (i, 0)),
                  pl.BlockSpec((BE, D_EDGE), lambda i: (i, 0)),
                  _full((D_EDGE, HQ)), _full((1, HQ)),
                  _full((1, HQ)), _full((1, 1))],
        out_specs=pl.BlockSpec((BE, 1), lambda i: (i, 0)),
        out_shape=jax.ShapeDtypeStruct((E, 1), f32),
    )(gsum, edge_attr, wc, cvec, w2, b2m)

    return mask.reshape(-1)
